# baseline (speedup 1.0000x reference)
import functools

import jax
import jax.numpy as jnp
from jax import lax
from jax.experimental import pallas as pl
from jax.experimental.pallas import tpu as pltpu
from jax.experimental.pallas import tpu_sc as plsc

BATCH = 1024
DIM = 16
VOCAB = 100000


def _sc_gather(table, idx):
    B = idx.shape[0]
    V, D = table.shape
    info = plsc.get_sparse_core_info()
    nw = info.num_cores * info.num_subcores
    b_per_w = B // nw

    mesh = plsc.VectorSubcoreMesh(core_axis_name="c", subcore_axis_name="s")

    @functools.partial(
        pl.kernel,
        mesh=mesh,
        out_type=jax.ShapeDtypeStruct((B, D), jnp.float32),
        scratch_types=[
            pltpu.VMEM((b_per_w,), jnp.int32),
            pltpu.VMEM((b_per_w, D), jnp.float32),
            pltpu.SemaphoreType.DMA,
        ],
        compiler_params=pltpu.CompilerParams(use_tc_tiling_on_sc=False),
    )
    def gather_kernel(table_hbm, idx_hbm, out_hbm, idx_v, rows_v, sem):
        wid = lax.axis_index("s") * info.num_cores + lax.axis_index("c")
        base = wid * b_per_w
        pltpu.sync_copy(idx_hbm.at[pl.ds(base, b_per_w)], idx_v)
        pltpu.async_copy(table_hbm.at[idx_v], rows_v, sem).wait()
        pltpu.sync_copy(rows_v, out_hbm.at[pl.ds(base, b_per_w)])

    return gather_kernel(table, idx)


VTILE = 2048


NOUT = 4


def _tc_project_t(x, Wt):
    B, D = x.shape
    V = Wt.shape[1]
    nv = pl.cdiv(V, VTILE)
    wlast = V - (nv - 1) * VTILE
    loff = (nv - 1) * VTILE

    def body(x_ref, w_ref, out_hbm, obuf, osem):
        i = pl.program_id(0)
        s = lax.rem(i, NOUT)

        @pl.when(i >= NOUT)
        def _wait_prev():
            pltpu.make_async_copy(
                obuf.at[s], out_hbm.at[pl.ds(0, VTILE), :], osem.at[s]
            ).wait()

        obuf[s] = lax.dot_general(
            w_ref[...], x_ref[...], (((0,), (1,)), ((), ())),
            preferred_element_type=jnp.float32,
        )

        @pl.when(i < nv - 1)
        def _store():
            pltpu.make_async_copy(
                obuf.at[s], out_hbm.at[pl.ds(i * VTILE, VTILE), :], osem.at[s]
            ).start()

        @pl.when(i == nv - 1)
        def _store_last_and_drain():
            pltpu.make_async_copy(
                obuf.at[s, pl.ds(0, wlast), :],
                out_hbm.at[pl.ds(loff, wlast), :],
                osem.at[s],
            ).start()
            for step in range(max(nv - NOUT, 0), nv):
                k = step % NOUT
                w = VTILE if step < nv - 1 else wlast
                pltpu.make_async_copy(
                    obuf.at[k, pl.ds(0, w), :],
                    out_hbm.at[pl.ds(0, w), :],
                    osem.at[k],
                ).wait()

    return pl.pallas_call(
        body,
        grid=(nv,),
        in_specs=[
            pl.BlockSpec((B, D), lambda i: (0, 0)),
            pl.BlockSpec((D, VTILE), lambda i: (0, i)),
        ],
        out_specs=pl.BlockSpec(memory_space=pl.ANY),
        out_shape=jax.ShapeDtypeStruct((V, B), jnp.float32),
        scratch_shapes=[
            pltpu.VMEM((NOUT, VTILE, B), jnp.float32),
            pltpu.SemaphoreType.DMA((NOUT,)),
        ],
    )(x, Wt)


@jax.jit
def kernel(input, embed, W, b):
    idx = input.astype(jnp.int32)
    x = _sc_gather(embed, idx)
    xb = jnp.concatenate(
        [x, jnp.ones((x.shape[0], 1), jnp.float32)], axis=1
    ).astype(jnp.bfloat16)
    wtb = jnp.concatenate([W.T, b[None, :]], axis=0).astype(jnp.bfloat16)
    return _tc_project_t(xb, wtb).T
